# Initial kernel scaffold; baseline (speedup 1.0000x reference)
import jax
import jax.numpy as jnp
from jax.experimental import pallas as pl


def kernel(node_features_in, edge_features_in, edges_indexes, enc_n_W1, enc_n_b1, enc_n_W2, enc_n_b2, enc_e_W1, enc_e_b1, enc_e_W2, enc_e_b2, proc_e_W1, proc_e_b1, proc_e_W2, proc_e_b2, proc_n_W1, proc_n_b1, proc_n_W2, proc_n_b2, dec_n_W1, dec_n_b1, dec_n_W2, dec_n_b2):
    raise NotImplementedError("write your pallas kernel here")


# speedup vs baseline: 3.7435x; 3.7435x over previous
import functools

import jax
import jax.numpy as jnp
from jax import lax
from jax.experimental import pallas as pl
from jax.experimental.pallas import tpu as pltpu
from jax.experimental.pallas import tpu_sc as plsc

_N = 10000
_E = 320000
_D = 128
_NC = 2
_NS = 16
_NW = _NC * _NS
_EPW = _E // _NW
_GCH = 80
_NCH = _EPW // _GCH
_RPW = 624

_mesh = plsc.VectorSubcoreMesh(core_axis_name="c", subcore_axis_name="s")


def _mlp2_body(x_ref, w1_ref, b1_ref, w2_ref, b2_ref, o_ref, *, residual):
    x = x_ref[...]
    h = jnp.dot(x, w1_ref[...], preferred_element_type=jnp.float32) + b1_ref[...]
    o = jnp.dot(h, w2_ref[...], preferred_element_type=jnp.float32) + b2_ref[...]
    if residual:
        o = o + x
    o_ref[...] = o


def _mlp2(x, W1, b1, W2, b2, blk, residual=False):
    R, K = x.shape
    Kh = W1.shape[1]
    Ko = W2.shape[1]
    return pl.pallas_call(
        functools.partial(_mlp2_body, residual=residual),
        grid=(R // blk,),
        in_specs=[
            pl.BlockSpec((blk, K), lambda i: (i, 0)),
            pl.BlockSpec((K, Kh), lambda i: (0, 0)),
            pl.BlockSpec((1, Kh), lambda i: (0, 0)),
            pl.BlockSpec((Kh, Ko), lambda i: (0, 0)),
            pl.BlockSpec((1, Ko), lambda i: (0, 0)),
        ],
        out_specs=pl.BlockSpec((blk, Ko), lambda i: (i, 0)),
        out_shape=jax.ShapeDtypeStruct((R, Ko), jnp.float32),
    )(x, W1, b1.reshape(1, -1), W2, b2.reshape(1, -1))


def _ab_body(x_ref, ws_ref, wd_ref, a_ref, b_ref):
    x = x_ref[...]
    a_ref[...] = jnp.dot(x, ws_ref[...], preferred_element_type=jnp.float32)
    b_ref[...] = jnp.dot(x, wd_ref[...], preferred_element_type=jnp.float32)


def _ab(h_n, Ws, Wd, blk=2000):
    return pl.pallas_call(
        _ab_body,
        grid=(_N // blk,),
        in_specs=[
            pl.BlockSpec((blk, _D), lambda i: (i, 0)),
            pl.BlockSpec((_D, _D), lambda i: (0, 0)),
            pl.BlockSpec((_D, _D), lambda i: (0, 0)),
        ],
        out_specs=[pl.BlockSpec((blk, _D), lambda i: (i, 0))] * 2,
        out_shape=[jax.ShapeDtypeStruct((_N, _D), jnp.float32)] * 2,
    )(h_n, Ws, Wd)


def _edge_body(ra_ref, rb_ref, he_ref, we_ref, w2_ref, b1_ref, b2_ref, o_ref):
    he = he_ref[...]
    h1 = (ra_ref[...] + rb_ref[...] + b1_ref[...]
          + jnp.dot(he, we_ref[...], preferred_element_type=jnp.float32))
    o_ref[...] = (jnp.dot(h1, w2_ref[...], preferred_element_type=jnp.float32)
                  + b2_ref[...] + he)


def _edge_step(ra, rb, h_e, We, W2, b1, b2, blk=4000):
    return pl.pallas_call(
        _edge_body,
        grid=(_E // blk,),
        in_specs=[
            pl.BlockSpec((blk, _D), lambda i: (i, 0)),
            pl.BlockSpec((blk, _D), lambda i: (i, 0)),
            pl.BlockSpec((blk, _D), lambda i: (i, 0)),
            pl.BlockSpec((_D, _D), lambda i: (0, 0)),
            pl.BlockSpec((_D, _D), lambda i: (0, 0)),
            pl.BlockSpec((1, _D), lambda i: (0, 0)),
            pl.BlockSpec((1, _D), lambda i: (0, 0)),
        ],
        out_specs=pl.BlockSpec((blk, _D), lambda i: (i, 0)),
        out_shape=jax.ShapeDtypeStruct((_E, _D), jnp.float32),
    )(ra, rb, h_e, We, W2, b1.reshape(1, -1), b2.reshape(1, -1))


def _node_body(hn_ref, p0_ref, p1_ref, w1a_ref, w1b_ref, b1_ref, w2_ref,
               b2_ref, o_ref):
    hn = hn_ref[...]
    agg = p0_ref[...] + p1_ref[...]
    h = (jnp.dot(hn, w1a_ref[...], preferred_element_type=jnp.float32)
         + jnp.dot(agg, w1b_ref[...], preferred_element_type=jnp.float32)
         + b1_ref[...])
    o_ref[...] = (jnp.dot(h, w2_ref[...], preferred_element_type=jnp.float32)
                  + b2_ref[...] + hn)


def _node_step(h_n, parts, W1a, W1b, b1, W2, b2, blk=1000):
    nb = _N // blk
    return pl.pallas_call(
        _node_body,
        grid=(nb,),
        in_specs=[
            pl.BlockSpec((blk, _D), lambda i: (i, 0)),
            pl.BlockSpec((blk, _D), lambda i: (i, 0)),
            pl.BlockSpec((blk, _D), lambda i, nb=nb: (i + nb, 0)),
            pl.BlockSpec((_D, _D), lambda i: (0, 0)),
            pl.BlockSpec((_D, _D), lambda i: (0, 0)),
            pl.BlockSpec((1, _D), lambda i: (0, 0)),
            pl.BlockSpec((_D, _D), lambda i: (0, 0)),
            pl.BlockSpec((1, _D), lambda i: (0, 0)),
        ],
        out_specs=pl.BlockSpec((blk, _D), lambda i: (i, 0)),
        out_shape=jax.ShapeDtypeStruct((_N, _D), jnp.float32),
    )(h_n, parts, parts, W1a, W1b, b1.reshape(1, -1), W2, b2.reshape(1, -1))


@functools.partial(
    pl.kernel,
    out_type=(jax.ShapeDtypeStruct((_E, _D), jnp.float32),
              jax.ShapeDtypeStruct((_E, _D), jnp.float32)),
    mesh=_mesh,
    scratch_types=[
        pltpu.VMEM((_NCH, _GCH), jnp.int32),
        pltpu.VMEM((_NCH, _GCH), jnp.int32),
        pltpu.VMEM((_GCH, _D), jnp.float32),
        pltpu.VMEM((_GCH, _D), jnp.float32),
        pltpu.VMEM((_GCH, _D), jnp.float32),
        pltpu.VMEM((_GCH, _D), jnp.float32),
        pltpu.SemaphoreType.DMA,
        pltpu.SemaphoreType.DMA,
    ],
)
def _gather_ab(a_hbm, b_hbm, src3_hbm, dst3_hbm, oa_hbm, ob_hbm,
               sidx, didx, ra0, rb0, ra1, rb1, g0, g1):
    wid = lax.axis_index("s") * _NC + lax.axis_index("c")
    base = wid * _EPW
    pltpu.sync_copy(src3_hbm.at[wid], sidx)
    pltpu.sync_copy(dst3_hbm.at[wid], didx)

    ras = (ra0, ra1)
    rbs = (rb0, rb1)
    gs = (g0, g1)

    def issue(j, slot):
        pltpu.async_copy(a_hbm.at[sidx.at[j]], ras[slot], gs[slot])
        pltpu.async_copy(b_hbm.at[didx.at[j]], rbs[slot], gs[slot])

    def drain_write(j, slot):
        pltpu.make_async_copy(a_hbm.at[sidx.at[j]], ras[slot], gs[slot]).wait()
        pltpu.make_async_copy(b_hbm.at[didx.at[j]], rbs[slot], gs[slot]).wait()
        off = base + j * _GCH
        pltpu.sync_copy(ras[slot], oa_hbm.at[pl.ds(off, _GCH)])
        pltpu.sync_copy(rbs[slot], ob_hbm.at[pl.ds(off, _GCH)])

    issue(0, 0)

    def body(jj, carry):
        j = jj * 2
        issue(j + 1, 1)
        drain_write(j, 0)
        issue(j + 2, 0)
        drain_write(j + 1, 1)
        return carry

    lax.fori_loop(0, (_NCH - 1) // 2, body, 0)
    drain_write(_NCH - 1, 0)


@functools.partial(
    pl.kernel,
    out_type=jax.ShapeDtypeStruct((2 * _N, _D), jnp.float32),
    mesh=_mesh,
    scratch_types=[
        pltpu.VMEM((_NCH, _GCH), jnp.int32),
        pltpu.VMEM((_GCH, _D), jnp.float32),
        pltpu.VMEM((_GCH, _D), jnp.float32),
        pltpu.VMEM_SHARED((_N, _D), jnp.float32),
        pltpu.SemaphoreType.DMA,
        pltpu.SemaphoreType.DMA,
    ],
)
def _scatter_dst(x_hbm, dst3_hbm, zeros_hbm, out_hbm, didx, r0, r1, acc,
                 s0, s1):
    c = lax.axis_index("c")
    s = lax.axis_index("s")
    wid = s * _NC + c
    base = wid * _EPW

    @pl.when(s == 0)
    def _():
        pltpu.sync_copy(zeros_hbm, acc)

    pltpu.sync_copy(dst3_hbm.at[wid], didx)
    plsc.subcore_barrier()

    rs = (r0, r1)
    ss = (s0, s1)

    def issue(j, slot):
        pltpu.async_copy(x_hbm.at[pl.ds(base + j * _GCH, _GCH)], rs[slot],
                         ss[slot])

    def drain_scatter(j, slot):
        pltpu.make_async_copy(x_hbm.at[pl.ds(0, _GCH)], rs[slot],
                              ss[slot]).wait()
        pltpu.sync_copy(rs[slot], acc.at[didx.at[j]], add=True)

    issue(0, 0)

    def body(jj, carry):
        j = jj * 2
        issue(j + 1, 1)
        drain_scatter(j, 0)
        issue(j + 2, 0)
        drain_scatter(j + 1, 1)
        return carry

    lax.fori_loop(0, (_NCH - 1) // 2, body, 0)
    drain_scatter(_NCH - 1, 0)

    plsc.subcore_barrier()

    @pl.when(s < _NS - 1)
    def _():
        pltpu.sync_copy(acc.at[pl.ds(s * _RPW, _RPW)],
                        out_hbm.at[pl.ds(c * _N + s * _RPW, _RPW)])

    @pl.when(s == _NS - 1)
    def _():
        tail = _N - (_NS - 1) * _RPW
        pltpu.sync_copy(acc.at[pl.ds((_NS - 1) * _RPW, tail)],
                        out_hbm.at[pl.ds(c * _N + (_NS - 1) * _RPW, tail)])


def kernel(node_features_in, edge_features_in, edges_indexes,
           enc_n_W1, enc_n_b1, enc_n_W2, enc_n_b2,
           enc_e_W1, enc_e_b1, enc_e_W2, enc_e_b2,
           proc_e_W1, proc_e_b1, proc_e_W2, proc_e_b2,
           proc_n_W1, proc_n_b1, proc_n_W2, proc_n_b2,
           dec_n_W1, dec_n_b1, dec_n_W2, dec_n_b2):
    src3 = edges_indexes[0].reshape(_NW, _NCH, _GCH)
    dst3 = edges_indexes[1].reshape(_NW, _NCH, _GCH)
    zeros = jnp.zeros((_N, _D), jnp.float32)

    h_n = _mlp2(node_features_in, enc_n_W1, enc_n_b1, enc_n_W2, enc_n_b2,
                blk=1000)
    h_e = _mlp2(edge_features_in, enc_e_W1, enc_e_b1, enc_e_W2, enc_e_b2,
                blk=4000)

    for t in range(4):
        Ws = proc_e_W1[t, :_D]
        Wd = proc_e_W1[t, _D:2 * _D]
        We = proc_e_W1[t, 2 * _D:]
        A, B = _ab(h_n, Ws, Wd)
        ra, rb = _gather_ab(A, B, src3, dst3)
        e_new = _edge_step(ra, rb, h_e, We, proc_e_W2[t], proc_e_b1[t],
                           proc_e_b2[t])
        parts = _scatter_dst(e_new, dst3, zeros)
        h_n = _node_step(h_n, parts, proc_n_W1[t, :_D], proc_n_W1[t, _D:],
                         proc_n_b1[t], proc_n_W2[t], proc_n_b2[t])
        h_e = e_new

    return _mlp2(h_n, dec_n_W1, dec_n_b1, dec_n_W2, dec_n_b2, blk=1000)
